# Initial kernel scaffold; baseline (speedup 1.0000x reference)
import jax
import jax.numpy as jnp
from jax.experimental import pallas as pl


def kernel(x, part_labels):
    raise NotImplementedError("write your pallas kernel here")


# speedup vs baseline: 48.4351x; 48.4351x over previous
import functools

import jax
import jax.numpy as jnp
from jax import lax
from jax.experimental import pallas as pl
from jax.experimental.pallas import tpu as pltpu
from jax.experimental.pallas import tpu_sc as plsc

P = 8
C = 128
HW = 2048
NROWS = 128
L = 16
VECS = HW // L
NW = 32
ROWS_PER_W = NROWS // NW


def _scalar(v16):
    return jnp.max(v16)


def _sc_pool(xf, labf):
    mesh = plsc.VectorSubcoreMesh(core_axis_name="c", subcore_axis_name="s")

    @functools.partial(
        pl.kernel,
        mesh=mesh,
        out_type=jax.ShapeDtypeStruct((NROWS, C * P), jnp.float32),
        scratch_types=[
            pltpu.VMEM((HW,), jnp.int32),
            pltpu.VMEM((P * (HW + L) + L,), jnp.int32),
            pltpu.VMEM((HW,), jnp.float32),
            pltpu.VMEM((HW,), jnp.float32),
            pltpu.VMEM((C * P * L,), jnp.float32),
            pltpu.VMEM((C * P * L,), jnp.float32),
            pltpu.VMEM((C * P,), jnp.float32),
            pltpu.SemaphoreType.DMA,
            pltpu.SemaphoreType.DMA,
            pltpu.SemaphoreType.DMA,
        ],
        compiler_params=pltpu.CompilerParams(needs_layout_passes=False),
    )
    def k(x_hbm, lab_hbm, out_hbm, lab_v, idx_buf, fb0, fb1, sum_st, max_st,
          out_v, sem0, sem1, semo):
        wid = lax.axis_index("s") * 2 + lax.axis_index("c")
        iota = lax.iota(jnp.int32, L)
        iota16 = iota * L
        lane_p = lax.rem(iota, P)

        for rr in range(ROWS_PER_W):
            r = wid * ROWS_PER_W + rr
            ni = lax.div(r, 16)
            si = lax.rem(r, 16)
            pltpu.sync_copy(lab_hbm.at[r], lab_v)

            trash = P * (HW + L)
            trash_idx = trash + iota

            def build_body(i, curs):
                lv = lab_v[pl.ds(i * L, L)]
                pix = iota + i * L
                new = []
                for p in range(P):
                    m = lv == p
                    mi = m.astype(jnp.int32)
                    rank = lax.cumsum(mi, axis=0) - 1
                    dest = jnp.where(m, p * (HW + L) + curs[p] + rank,
                                     trash_idx)
                    plsc.store_scatter(idx_buf, [dest], pix)
                    new.append(curs[p] +
                               _scalar(plsc.all_reduce_population_count(m)))
                return tuple(new)

            counts = lax.fori_loop(0, VECS, build_body, (0,) * P)

            c16 = jnp.zeros((L,), jnp.float32)
            for p in range(P):
                c16 = jnp.where(lane_p == p,
                                jnp.full((L,), counts[p]).astype(jnp.float32),
                                c16)
            nfull = [counts[p] // L for p in range(P)]
            rem = [counts[p] - nfull[p] * L for p in range(P)]

            def chan_flat(ch):
                return (ni * C + ch) * 16 + si

            def start_feat(ch, buf, sem):
                pltpu.make_async_copy(x_hbm.at[chan_flat(ch)], buf, sem).start()

            def wait_feat(ch, buf, sem):
                pltpu.make_async_copy(x_hbm.at[chan_flat(ch)], buf, sem).wait()

            start_feat(0, fb0, sem0)
            start_feat(1, fb1, sem1)

            def do_channel(ch, fb):
                for p in range(P):
                    def gat_body(j, accs):
                        s_a, m_a = accs
                        iv = idx_buf[pl.ds(p * (HW + L) + j * L, L)]
                        v = plsc.load_gather(fb, [iv])
                        return s_a + v, jnp.maximum(m_a, v)

                    acc = (jnp.zeros((L,), jnp.float32),
                           jnp.full((L,), -100.0, jnp.float32))
                    acc_s, acc_m = lax.fori_loop(0, nfull[p], gat_body, acc)
                    mt = iota < rem[p]
                    ivr = idx_buf[pl.ds(p * (HW + L) + nfull[p] * L, L)]
                    ivc = jnp.where(mt, ivr, 0)
                    vt = plsc.load_gather(fb, [ivc])
                    acc_s = acc_s + jnp.where(mt, vt, 0.0)
                    acc_m = jnp.maximum(acc_m,
                                        jnp.where(mt, vt, -100.0))
                    base = (ch * P + p) * L
                    sum_st[pl.ds(base, L)] = acc_s
                    max_st[pl.ds(base, L)] = acc_m

            def chan_body(i, _):
                ch0 = i * 2
                wait_feat(ch0, fb0, sem0)
                do_channel(ch0, fb0)

                @pl.when(ch0 + 2 < C)
                def _():
                    start_feat(ch0 + 2, fb0, sem0)

                wait_feat(ch0 + 1, fb1, sem1)
                do_channel(ch0 + 1, fb1)

                @pl.when(ch0 + 3 < C)
                def _():
                    start_feat(ch0 + 3, fb1, sem1)

                return 0

            lax.fori_loop(0, C // 2, chan_body, 0)

            def fin_body(g, _):
                base = g * (L * L)
                s_a = jnp.zeros((L,), jnp.float32)
                m_a = jnp.full((L,), -100.0, jnp.float32)
                for j in range(L):
                    idxv = iota16 + (base + j)
                    s_a = s_a + plsc.load_gather(sum_st, [idxv])
                    m_a = jnp.maximum(m_a, plsc.load_gather(max_st, [idxv]))
                mean = s_a / jnp.maximum(c16, 1.0)
                mx = jnp.where(c16 > 0.0, m_a, 0.0)
                out_v[pl.ds(g * L, L)] = mean + mx
                return 0

            lax.fori_loop(0, (C * P) // L, fin_body, 0)

            pltpu.make_async_copy(out_v, out_hbm.at[r], semo).start()
            pltpu.make_async_copy(out_v, out_hbm.at[r], semo).wait()

    return k(xf, labf)


def kernel(x, part_labels):
    n, c, s, h, w = x.shape
    xf = x.reshape(n * c * s, h * w)
    labf = part_labels.reshape(n * s, h * w).astype(jnp.int32)
    pooled = _sc_pool(xf, labf)
    return pooled.reshape(n, s, c, P).transpose(0, 2, 1, 3)


# speedup vs baseline: 66.8825x; 1.3809x over previous
import functools

import jax
import jax.numpy as jnp
from jax import lax
from jax.experimental import pallas as pl
from jax.experimental.pallas import tpu as pltpu
from jax.experimental.pallas import tpu_sc as plsc

P = 8
C = 128
HW = 2048
NROWS = 128
L = 16
VECS = HW // L
NW = 32
ROWS_PER_W = NROWS // NW
SEG = HW + L


def _sc_pool(xf, labf):
    mesh = plsc.VectorSubcoreMesh(core_axis_name="c", subcore_axis_name="s")

    @functools.partial(
        pl.kernel,
        mesh=mesh,
        out_type=jax.ShapeDtypeStruct((NROWS, C * P), jnp.float32),
        scratch_types=[
            pltpu.VMEM((HW,), jnp.int32),
            pltpu.VMEM((P * SEG + L,), jnp.int32),
            pltpu.VMEM((HW,), jnp.float32),
            pltpu.VMEM((HW,), jnp.float32),
            pltpu.VMEM((HW,), jnp.float32),
            pltpu.VMEM((HW,), jnp.float32),
            pltpu.VMEM((C * P * L,), jnp.float32),
            pltpu.VMEM((C * P * L,), jnp.float32),
            pltpu.VMEM((C * P,), jnp.float32),
            pltpu.SemaphoreType.DMA,
            pltpu.SemaphoreType.DMA,
            pltpu.SemaphoreType.DMA,
            pltpu.SemaphoreType.DMA,
            pltpu.SemaphoreType.DMA,
        ],
        compiler_params=pltpu.CompilerParams(needs_layout_passes=False),
    )
    def k(x_hbm, lab_hbm, out_hbm, lab_v, idx_buf, fa0, fa1, fb0, fb1,
          sum_st, max_st, out_v, sa0, sa1, sb0, sb1, semo):
        wid = lax.axis_index("s") * 2 + lax.axis_index("c")
        iota = lax.iota(jnp.int32, L)
        iota16 = iota * L
        lane_p = lax.rem(iota, P)
        trash_idx = P * SEG + iota

        for rr in range(ROWS_PER_W):
            r = wid * ROWS_PER_W + rr
            ni = lax.div(r, 16)
            si = lax.rem(r, 16)
            pltpu.sync_copy(lab_hbm.at[r], lab_v)

            zero = jnp.zeros((L,), jnp.int32)

            @plsc.parallel_loop(0, VECS, unroll=2, carry=(zero,) * P)
            def _build(i, curs):
                lv = lab_v[pl.ds(i * L, L)]
                pix = iota + i * L
                new = []
                for p in range(P):
                    m = lv == p
                    mi = m.astype(jnp.int32)
                    rank = lax.cumsum(mi, axis=0) - 1
                    dest = jnp.where(m, p * SEG + curs[p] + rank, trash_idx)
                    plsc.store_scatter(idx_buf, [dest], pix)
                    new.append(curs[p] + plsc.all_reduce_population_count(m))
                return tuple(new)

            counts = [jnp.max(cv) for cv in _build]

            c16 = jnp.zeros((L,), jnp.float32)
            for p in range(P):
                c16 = jnp.where(lane_p == p,
                                jnp.full((L,), counts[p]).astype(jnp.float32),
                                c16)
            nfull = [counts[p] // L for p in range(P)]
            rem = [counts[p] - nfull[p] * L for p in range(P)]

            def chan_flat(ch):
                return (ni * C + ch) * 16 + si

            def start_feat(ch, buf, sem):
                pltpu.make_async_copy(x_hbm.at[chan_flat(ch)], buf, sem).start()

            def wait_feat(ch, buf, sem):
                pltpu.make_async_copy(x_hbm.at[chan_flat(ch)], buf, sem).wait()

            start_feat(0, fa0, sa0)
            start_feat(1, fa1, sa1)
            start_feat(2, fb0, sb0)
            start_feat(3, fb1, sb1)
            sets = ((fa0, fa1, sa0, sa1), (fb0, fb1, sb0, sb1))

            def do_pair(c0, b0, b1):
                for p in range(P):
                    pb = p * SEG
                    init = (jnp.zeros((L,), jnp.float32),
                            jnp.full((L,), -100.0, jnp.float32),
                            jnp.zeros((L,), jnp.float32),
                            jnp.full((L,), -100.0, jnp.float32))

                    @plsc.parallel_loop(0, nfull[p], unroll=4, carry=init)
                    def _gat(j, acc):
                        s0, m0, s1, m1 = acc
                        iv = idx_buf[pl.ds(pb + j * L, L)]
                        v0 = plsc.load_gather(b0, [iv])
                        v1 = plsc.load_gather(b1, [iv])
                        return (s0 + v0, jnp.maximum(m0, v0),
                                s1 + v1, jnp.maximum(m1, v1))

                    s0, m0, s1, m1 = _gat
                    mt = iota < rem[p]
                    ivr = idx_buf[pl.ds(pb + nfull[p] * L, L)]
                    ivc = jnp.where(mt, ivr, 0)
                    v0 = plsc.load_gather(b0, [ivc])
                    v1 = plsc.load_gather(b1, [ivc])
                    s0 = s0 + jnp.where(mt, v0, 0.0)
                    m0 = jnp.maximum(m0, jnp.where(mt, v0, -100.0))
                    s1 = s1 + jnp.where(mt, v1, 0.0)
                    m1 = jnp.maximum(m1, jnp.where(mt, v1, -100.0))
                    base0 = (c0 * P + p) * L
                    base1 = ((c0 + 1) * P + p) * L
                    sum_st[pl.ds(base0, L)] = s0
                    max_st[pl.ds(base0, L)] = m0
                    sum_st[pl.ds(base1, L)] = s1
                    max_st[pl.ds(base1, L)] = m1

            def pair_body(i, _):
                for s_i, (b0, b1, s0, s1) in enumerate(sets):
                    pi = i * 2 + s_i
                    c0 = pi * 2
                    wait_feat(c0, b0, s0)
                    wait_feat(c0 + 1, b1, s1)
                    do_pair(c0, b0, b1)

                    @pl.when(c0 + 4 < C)
                    def _():
                        start_feat(c0 + 4, b0, s0)
                        start_feat(c0 + 5, b1, s1)

                return 0

            lax.fori_loop(0, C // 4, pair_body, 0)

            @plsc.parallel_loop(0, (C * P) // L, unroll=2)
            def _fin(g):
                base = g * (L * L)
                s_a = jnp.zeros((L,), jnp.float32)
                m_a = jnp.full((L,), -100.0, jnp.float32)
                for j in range(L):
                    idxv = iota16 + (base + j)
                    s_a = s_a + plsc.load_gather(sum_st, [idxv])
                    m_a = jnp.maximum(m_a, plsc.load_gather(max_st, [idxv]))
                mean = s_a / jnp.maximum(c16, 1.0)
                mx = jnp.where(c16 > 0.0, m_a, 0.0)
                out_v[pl.ds(g * L, L)] = mean + mx

            pltpu.make_async_copy(out_v, out_hbm.at[r], semo).start()
            pltpu.make_async_copy(out_v, out_hbm.at[r], semo).wait()

    return k(xf, labf)


def kernel(x, part_labels):
    n, c, s, h, w = x.shape
    xf = x.reshape(n * c * s, h * w)
    labf = part_labels.reshape(n * s, h * w).astype(jnp.int32)
    pooled = _sc_pool(xf, labf)
    return pooled.reshape(n, s, c, P).transpose(0, 2, 1, 3)


# speedup vs baseline: 67.1883x; 1.0046x over previous
import functools

import jax
import jax.numpy as jnp
from jax import lax
from jax.experimental import pallas as pl
from jax.experimental.pallas import tpu as pltpu
from jax.experimental.pallas import tpu_sc as plsc

P = 8
C = 128
HW = 2048
NROWS = 128
L = 16
VECS = HW // L
NW = 32
ROWS_PER_W = NROWS // NW
SEG = HW + L


def _sc_pool(xf, labf):
    mesh = plsc.VectorSubcoreMesh(core_axis_name="c", subcore_axis_name="s")

    @functools.partial(
        pl.kernel,
        mesh=mesh,
        out_type=jax.ShapeDtypeStruct((NROWS, C * P), jnp.float32),
        scratch_types=[
            pltpu.VMEM((64, 32), jnp.int32),
            pltpu.VMEM((P * SEG + L,), jnp.int32),
            pltpu.VMEM((64, 32), jnp.float32),
            pltpu.VMEM((64, 32), jnp.float32),
            pltpu.VMEM((64, 32), jnp.float32),
            pltpu.VMEM((64, 32), jnp.float32),
            pltpu.VMEM((C * P * L,), jnp.float32),
            pltpu.VMEM((C * P * L,), jnp.float32),
            pltpu.VMEM((C * P,), jnp.float32),
            pltpu.SemaphoreType.DMA,
            pltpu.SemaphoreType.DMA,
            pltpu.SemaphoreType.DMA,
            pltpu.SemaphoreType.DMA,
            pltpu.SemaphoreType.DMA,
        ],
        compiler_params=pltpu.CompilerParams(needs_layout_passes=False),
    )
    def k(x_hbm, lab_hbm, out_hbm, lab_v, idx_buf, fa0, fa1, fb0, fb1,
          sum_st, max_st, out_v, sa0, sa1, sb0, sb1, semo):
        wid = lax.axis_index("s") * 2 + lax.axis_index("c")
        iota = lax.iota(jnp.int32, L)
        iota16 = iota * L
        lane_p = lax.rem(iota, P)
        trash_idx = P * SEG + iota

        for rr in range(ROWS_PER_W):
            r = wid * ROWS_PER_W + rr
            ni = lax.div(r, 16)
            si = lax.rem(r, 16)
            pltpu.sync_copy(lab_hbm.at[ni, si], lab_v)

            zero = jnp.zeros((L,), jnp.int32)

            @plsc.parallel_loop(0, VECS, unroll=2, carry=(zero,) * P)
            def _build(i, curs):
                lrow = lax.div(i, 2)
                lcol = lax.rem(i, 2) * L
                lv = lab_v[lrow, pl.ds(lcol, L)]
                pix = iota + i * L
                new = []
                for p in range(P):
                    m = lv == p
                    mi = m.astype(jnp.int32)
                    rank = lax.cumsum(mi, axis=0) - 1
                    dest = jnp.where(m, p * SEG + curs[p] + rank, trash_idx)
                    plsc.store_scatter(idx_buf, [dest], pix)
                    new.append(curs[p] + plsc.all_reduce_population_count(m))
                return tuple(new)

            counts = [jnp.max(cv) for cv in _build]

            c16 = jnp.zeros((L,), jnp.float32)
            for p in range(P):
                c16 = jnp.where(lane_p == p,
                                jnp.full((L,), counts[p]).astype(jnp.float32),
                                c16)
            nfull = [counts[p] // L for p in range(P)]
            rem = [counts[p] - nfull[p] * L for p in range(P)]

            def start_feat(ch, buf, sem):
                pltpu.make_async_copy(x_hbm.at[ni, ch, si], buf, sem).start()

            def wait_feat(ch, buf, sem):
                pltpu.make_async_copy(x_hbm.at[ni, ch, si], buf, sem).wait()

            start_feat(0, fa0, sa0)
            start_feat(1, fa1, sa1)
            start_feat(2, fb0, sb0)
            start_feat(3, fb1, sb1)
            sets = ((fa0, fa1, sa0, sa1), (fb0, fb1, sb0, sb1))

            def do_pair(c0, b0, b1):
                for p in range(P):
                    pb = p * SEG
                    init = (jnp.zeros((L,), jnp.float32),
                            jnp.full((L,), -100.0, jnp.float32),
                            jnp.zeros((L,), jnp.float32),
                            jnp.full((L,), -100.0, jnp.float32))

                    @plsc.parallel_loop(0, nfull[p], unroll=4, carry=init)
                    def _gat(j, acc):
                        s0, m0, s1, m1 = acc
                        iv = idx_buf[pl.ds(pb + j * L, L)]
                        ivr = lax.shift_right_logical(iv, 5)
                        ivcl = lax.bitwise_and(iv, 31)
                        v0 = plsc.load_gather(b0, [ivr, ivcl])
                        v1 = plsc.load_gather(b1, [ivr, ivcl])
                        return (s0 + v0, jnp.maximum(m0, v0),
                                s1 + v1, jnp.maximum(m1, v1))

                    s0, m0, s1, m1 = _gat
                    mt = iota < rem[p]
                    ivt = idx_buf[pl.ds(pb + nfull[p] * L, L)]
                    ivc = jnp.where(mt, ivt, 0)
                    tr = lax.shift_right_logical(ivc, 5)
                    tc = lax.bitwise_and(ivc, 31)
                    v0 = plsc.load_gather(b0, [tr, tc])
                    v1 = plsc.load_gather(b1, [tr, tc])
                    s0 = s0 + jnp.where(mt, v0, 0.0)
                    m0 = jnp.maximum(m0, jnp.where(mt, v0, -100.0))
                    s1 = s1 + jnp.where(mt, v1, 0.0)
                    m1 = jnp.maximum(m1, jnp.where(mt, v1, -100.0))
                    base0 = (c0 * P + p) * L
                    base1 = ((c0 + 1) * P + p) * L
                    sum_st[pl.ds(base0, L)] = s0
                    max_st[pl.ds(base0, L)] = m0
                    sum_st[pl.ds(base1, L)] = s1
                    max_st[pl.ds(base1, L)] = m1

            def pair_body(i, _):
                for s_i, (b0, b1, s0, s1) in enumerate(sets):
                    pi = i * 2 + s_i
                    c0 = pi * 2
                    wait_feat(c0, b0, s0)
                    wait_feat(c0 + 1, b1, s1)
                    do_pair(c0, b0, b1)

                    @pl.when(c0 + 4 < C)
                    def _():
                        start_feat(c0 + 4, b0, s0)
                        start_feat(c0 + 5, b1, s1)

                return 0

            lax.fori_loop(0, C // 4, pair_body, 0)

            @plsc.parallel_loop(0, (C * P) // L, unroll=2)
            def _fin(g):
                base = g * (L * L)
                s_a = jnp.zeros((L,), jnp.float32)
                m_a = jnp.full((L,), -100.0, jnp.float32)
                for j in range(L):
                    idxv = iota16 + (base + j)
                    s_a = s_a + plsc.load_gather(sum_st, [idxv])
                    m_a = jnp.maximum(m_a, plsc.load_gather(max_st, [idxv]))
                mean = s_a / jnp.maximum(c16, 1.0)
                mx = jnp.where(c16 > 0.0, m_a, 0.0)
                out_v[pl.ds(g * L, L)] = mean + mx

            pltpu.make_async_copy(out_v, out_hbm.at[r], semo).start()
            pltpu.make_async_copy(out_v, out_hbm.at[r], semo).wait()

    return k(xf, labf)


def kernel(x, part_labels):
    n, c, s, h, w = x.shape
    pooled = _sc_pool(x, part_labels.astype(jnp.int32))
    return pooled.reshape(n, s, c, P).transpose(0, 2, 1, 3)
